# Initial kernel scaffold; baseline (speedup 1.0000x reference)
import jax
import jax.numpy as jnp
from jax.experimental import pallas as pl


def kernel(x, edge_index, W, b):
    raise NotImplementedError("write your pallas kernel here")


# speedup vs baseline: 15.3717x; 15.3717x over previous
import functools

import jax
import jax.numpy as jnp
from jax import lax
from jax.experimental import pallas as pl
from jax.experimental.pallas import tpu as pltpu
from jax.experimental.pallas import tpu_sc as plsc

N_NODES = 10000
N_EDGES = 320000
DIM = 128

NC = 2
NS = 16
NW = NC * NS
NPAD = 10240
RPT = NPAD // NS
C = 128
E_MSG = N_EDGES // NW
E_DEG = N_EDGES // NS
MSG_CHUNKS = E_MSG // C
MSG_TAIL = E_MSG - MSG_CHUNKS * C
DEG_CHUNKS = E_DEG // C
DEG_TAIL = E_DEG - DEG_CHUNKS * C

_MM_BLOCK = 400


def _mm_body(x_ref, w_ref, o_ref):
    o_ref[...] = jnp.dot(x_ref[...], w_ref[...],
                         preferred_element_type=jnp.float32)


def _final_body(p0_ref, p1_ref, xw_ref, dis_ref, b_ref, o_ref):
    dis = dis_ref[...]
    acc = (p0_ref[0] + p1_ref[0] + dis * dis * xw_ref[...] + b_ref[...])
    o_ref[...] = jnp.maximum(acc, 0.0)


def _rsqrt16(x):
    i = lax.bitcast_convert_type(x, jnp.int32)
    y = lax.bitcast_convert_type(
        jnp.full((16,), 0x5F3759DF, jnp.int32) - (i >> 1), jnp.float32)
    for _ in range(3):
        y = y * (1.5 - 0.5 * x * y * y)
    return y


def _sc_body(src_hbm, dst_hbm, xw_hbm,
             part_hbm, dis_hbm,
             hist_sh, dis_sh, acc_sh,
             dis_v, src_v, dst_v, norm_v, rows_v, ones_v, slice_v, zero_v,
             srct_v, dstt_v, dstt2_v, ones32_v, rowst_v, normt_v, sem):
    cid = lax.axis_index("c")
    sid = lax.axis_index("s")
    z16 = jnp.zeros((16,), jnp.float32)
    o16 = jnp.ones((16,), jnp.float32)
    row0 = sid * RPT

    for j in range(C // 16):
        ones_v[pl.ds(j * 16, 16)] = o16
    for j in range(2):
        ones32_v[pl.ds(j * 16, 16)] = o16
    for j in range(RPT // 16):
        slice_v[pl.ds(j * 16, 16)] = z16

    @pl.loop(0, C)
    def _zero_rows(i):
        for j in range(DIM // 16):
            zero_v[i, pl.ds(j * 16, 16)] = z16

    pltpu.sync_copy(slice_v, hist_sh.at[pl.ds(row0, RPT)])
    for k in range(RPT // C):
        pltpu.sync_copy(zero_v, acc_sh.at[pl.ds(row0 + k * C, C)])
    plsc.subcore_barrier()

    dbase = sid * E_DEG

    @pl.loop(0, DEG_CHUNKS)
    def _deg(ci):
        b = dbase + ci * C
        pltpu.sync_copy(dst_hbm.at[pl.ds(b, C)], dst_v)
        pltpu.sync_copy(ones_v, hist_sh.at[dst_v], add=True)

    pltpu.sync_copy(dst_hbm.at[pl.ds(dbase + DEG_CHUNKS * C, DEG_TAIL)],
                    dstt2_v)
    pltpu.sync_copy(ones32_v, hist_sh.at[dstt2_v], add=True)
    plsc.subcore_barrier()

    pltpu.sync_copy(hist_sh.at[pl.ds(row0, RPT)], slice_v)
    for j in range(RPT // 16):
        deg = slice_v[pl.ds(j * 16, 16)] + 1.0
        slice_v[pl.ds(j * 16, 16)] = _rsqrt16(deg)
    pltpu.sync_copy(slice_v, dis_sh.at[pl.ds(row0, RPT)])

    @pl.when(cid == 0)
    def _dis_out():
        pltpu.sync_copy(slice_v, dis_hbm.at[pl.ds(row0, RPT)])

    plsc.subcore_barrier()
    pltpu.sync_copy(dis_sh, dis_v)

    wid = cid * NS + sid
    mbase = wid * E_MSG

    @pl.loop(0, MSG_CHUNKS)
    def _msg(ci):
        b = mbase + ci * C
        pltpu.sync_copy(src_hbm.at[pl.ds(b, C)], src_v)
        pltpu.sync_copy(dst_hbm.at[pl.ds(b, C)], dst_v)
        for g in range(C // 16):
            sv = src_v[pl.ds(g * 16, 16)]
            dv = dst_v[pl.ds(g * 16, 16)]
            ns = plsc.load_gather(dis_v, [sv])
            nd = plsc.load_gather(dis_v, [dv])
            norm_v[pl.ds(g * 16, 16)] = ns * nd
        pltpu.async_copy(xw_hbm.at[src_v], rows_v, sem).wait()

        @pl.loop(0, C)
        def _scale(r):
            idx = jnp.broadcast_to(r, (16,)).astype(jnp.int32)
            spl = plsc.load_gather(norm_v, [idx])
            for j in range(DIM // 16):
                rows_v[r, pl.ds(j * 16, 16)] = (
                    rows_v[r, pl.ds(j * 16, 16)] * spl)

        pltpu.sync_copy(rows_v, acc_sh.at[dst_v], add=True)

    tb = mbase + MSG_CHUNKS * C
    pltpu.sync_copy(src_hbm.at[pl.ds(tb, MSG_TAIL)], srct_v)
    pltpu.sync_copy(dst_hbm.at[pl.ds(tb, MSG_TAIL)], dstt_v)
    normt_v[...] = (plsc.load_gather(dis_v, [srct_v[...]]) *
                    plsc.load_gather(dis_v, [dstt_v[...]]))
    pltpu.async_copy(xw_hbm.at[srct_v], rowst_v, sem).wait()

    @pl.loop(0, MSG_TAIL)
    def _scale_tail(r):
        idx = jnp.broadcast_to(r, (16,)).astype(jnp.int32)
        spl = plsc.load_gather(normt_v, [idx])
        for j in range(DIM // 16):
            rowst_v[r, pl.ds(j * 16, 16)] = (
                rowst_v[r, pl.ds(j * 16, 16)] * spl)

    pltpu.sync_copy(rowst_v, acc_sh.at[dstt_v], add=True)
    plsc.subcore_barrier()

    pltpu.sync_copy(acc_sh.at[pl.ds(row0, RPT)],
                    part_hbm.at[cid, pl.ds(row0, RPT)])


_sc_kernel = functools.partial(
    pl.kernel,
    out_type=(
        jax.ShapeDtypeStruct((NC, NPAD, DIM), jnp.float32),
        jax.ShapeDtypeStruct((NPAD,), jnp.float32),
    ),
    mesh=plsc.VectorSubcoreMesh(core_axis_name="c", subcore_axis_name="s",
                                num_cores=NC, num_subcores=NS),
    compiler_params=pltpu.CompilerParams(needs_layout_passes=False),
    scratch_types=[
        pltpu.VMEM_SHARED((NPAD,), jnp.float32),
        pltpu.VMEM_SHARED((NPAD,), jnp.float32),
        pltpu.VMEM_SHARED((NPAD, DIM), jnp.float32),
        pltpu.VMEM((NPAD,), jnp.float32),
        pltpu.VMEM((C,), jnp.int32),
        pltpu.VMEM((C,), jnp.int32),
        pltpu.VMEM((C,), jnp.float32),
        pltpu.VMEM((C, DIM), jnp.float32),
        pltpu.VMEM((C,), jnp.float32),
        pltpu.VMEM((RPT,), jnp.float32),
        pltpu.VMEM((C, DIM), jnp.float32),
        pltpu.VMEM((MSG_TAIL,), jnp.int32),
        pltpu.VMEM((MSG_TAIL,), jnp.int32),
        pltpu.VMEM((DEG_TAIL,), jnp.int32),
        pltpu.VMEM((DEG_TAIL,), jnp.float32),
        pltpu.VMEM((MSG_TAIL, DIM), jnp.float32),
        pltpu.VMEM((MSG_TAIL,), jnp.float32),
        pltpu.SemaphoreType.DMA,
    ],
)(_sc_body)


def kernel(x, edge_index, W, b):
    ei = edge_index.astype(jnp.int32)
    src = ei[0]
    dst = ei[1]

    xw = pl.pallas_call(
        _mm_body,
        grid=(N_NODES // _MM_BLOCK,),
        in_specs=[
            pl.BlockSpec((_MM_BLOCK, DIM), lambda i: (i, 0)),
            pl.BlockSpec((DIM, DIM), lambda i: (0, 0)),
        ],
        out_specs=pl.BlockSpec((_MM_BLOCK, DIM), lambda i: (i, 0)),
        out_shape=jax.ShapeDtypeStruct((N_NODES, DIM), jnp.float32),
    )(x, W)

    part, dis = _sc_kernel(src, dst, xw)

    dis2d = dis[:N_NODES].reshape(N_NODES, 1)
    b2d = b.reshape(1, DIM)
    out = pl.pallas_call(
        _final_body,
        grid=(N_NODES // _MM_BLOCK,),
        in_specs=[
            pl.BlockSpec((1, _MM_BLOCK, DIM), lambda i: (0, i, 0)),
            pl.BlockSpec((1, _MM_BLOCK, DIM), lambda i: (1, i, 0)),
            pl.BlockSpec((_MM_BLOCK, DIM), lambda i: (i, 0)),
            pl.BlockSpec((_MM_BLOCK, 1), lambda i: (i, 0)),
            pl.BlockSpec((1, DIM), lambda i: (0, 0)),
        ],
        out_specs=pl.BlockSpec((_MM_BLOCK, DIM), lambda i: (i, 0)),
        out_shape=jax.ShapeDtypeStruct((N_NODES, DIM), jnp.float32),
    )(part, part, xw, dis2d, b2d)
    return out


# speedup vs baseline: 23.7044x; 1.5421x over previous
import functools

import jax
import jax.numpy as jnp
from jax import lax
from jax.experimental import pallas as pl
from jax.experimental.pallas import tpu as pltpu
from jax.experimental.pallas import tpu_sc as plsc

N_NODES = 10000
N_EDGES = 320000
DIM = 128

NC = 2
NS = 16
NW = NC * NS
NPAD = 10240
RPT = NPAD // NS
C = 128
E_MSG = N_EDGES // NW
MSG_CHUNKS = E_MSG // C
MSG_TAIL = E_MSG - MSG_CHUNKS * C

E_DEG = N_EDGES // NS
DEG_CHUNKS = E_DEG // C
DEG_TAIL = E_DEG - DEG_CHUNKS * C

_MM_BLOCK = 400


def _mm_body(x_ref, w_ref, o_ref):
    o_ref[...] = jnp.dot(x_ref[...], w_ref[...],
                         preferred_element_type=jnp.float32)


def _final_body(p0_ref, p1_ref, xw_ref, dis_ref, b_ref, o_ref):
    dis = dis_ref[...]
    acc = (p0_ref[0] + p1_ref[0] + dis * dis * xw_ref[...] + b_ref[...])
    o_ref[...] = jnp.maximum(acc, 0.0)


def _rsqrt16(x):
    i = lax.bitcast_convert_type(x, jnp.int32)
    y = lax.bitcast_convert_type(
        jnp.full((16,), 0x5F3759DF, jnp.int32) - (i >> 1), jnp.float32)
    for _ in range(3):
        y = y * (1.5 - 0.5 * x * y * y)
    return y


def _sc_body(src_hbm, dst_hbm, xw_hbm,
             part_hbm, dis_hbm,
             hist_sh, acc_sh,
             dis_v,
             src_b0, src_b1, src_b2, dst_b0, dst_b1, dst_b2,
             norm_b0, norm_b1, norm_b2, rows_b0, rows_b1,
             ones_v, ones32_v, slice_v, srct_v, dstt_v, dstt2_v,
             normt_v,
             sem_i0, sem_i1, sem_i2, sem_r0, sem_r1,
             sem_s0, sem_s1, sem_s2, sem_x):
    cid = lax.axis_index("c")
    sid = lax.axis_index("s")
    z16 = jnp.zeros((16,), jnp.float32)
    o16 = jnp.ones((16,), jnp.float32)
    row0 = sid * RPT

    src_bufs = [src_b0, src_b1, src_b2]
    dst_bufs = [dst_b0, dst_b1, dst_b2]
    norm_bufs = [norm_b0, norm_b1, norm_b2]
    rows_bufs = [rows_b0, rows_b1]
    sem_idx = [sem_i0, sem_i1, sem_i2]
    sem_rows = [sem_r0, sem_r1]
    sem_scat = [sem_s0, sem_s1, sem_s2]

    for j in range(C // 16):
        ones_v[pl.ds(j * 16, 16)] = o16
    for j in range(DEG_TAIL // 16):
        ones32_v[pl.ds(j * 16, 16)] = o16
    for j in range(RPT // 16):
        slice_v[pl.ds(j * 16, 16)] = z16

    @pl.loop(0, C)
    def _zero_rows(i):
        for j in range(DIM // 16):
            rows_b0[i, pl.ds(j * 16, 16)] = z16

    pltpu.sync_copy(slice_v, hist_sh.at[pl.ds(row0, RPT)])
    for k in range(RPT // C):
        pltpu.sync_copy(rows_b0, acc_sh.at[pl.ds(row0 + k * C, C)])
    plsc.subcore_barrier()

    dgbase = sid * E_DEG

    def deg_stage(ci, bn):
        pltpu.async_copy(dst_hbm.at[pl.ds(dgbase + ci * C, C)],
                         dst_bufs[bn], sem_idx[bn])

    def deg_body(ci, b, bn, wait_scat, do_stage):
        if wait_scat:
            pltpu.make_async_copy(dst_hbm.at[pl.ds(0, C)], dst_bufs[bn],
                                  sem_scat[bn]).wait()
        if do_stage is True:
            deg_stage(ci + 1, bn)
        elif do_stage is not False:
            @pl.when(do_stage)
            def _():
                deg_stage(ci + 1, bn)
        pltpu.make_async_copy(dst_hbm.at[pl.ds(0, C)], dst_bufs[b],
                              sem_idx[b]).wait()
        pltpu.async_copy(ones_v, hist_sh.at[dst_bufs[b]], sem_scat[b],
                         add=True)

    deg_stage(0, 0)
    deg_body(0, 0, 1, wait_scat=False, do_stage=True)
    deg_body(1, 1, 2, wait_scat=False, do_stage=True)
    deg_body(2, 2, 0, wait_scat=True, do_stage=True)

    @pl.loop(1, DEG_CHUNKS // 3)
    def _deg(si):
        ci0 = si * 3
        for k in range(3):
            ci = ci0 + k
            deg_body(ci, k, (k + 1) % 3, wait_scat=True,
                     do_stage=ci + 1 < DEG_CHUNKS)

    for b in (1, 2):
        pltpu.make_async_copy(dst_hbm.at[pl.ds(0, C)], dst_bufs[b],
                              sem_scat[b]).wait()
    pltpu.sync_copy(dst_hbm.at[pl.ds(dgbase + DEG_CHUNKS * C, DEG_TAIL)],
                    dstt2_v)
    pltpu.sync_copy(ones32_v, hist_sh.at[dstt2_v], add=True)
    plsc.subcore_barrier()

    pltpu.sync_copy(hist_sh.at[pl.ds(row0, RPT)], slice_v)
    for j in range(RPT // 16):
        deg = slice_v[pl.ds(j * 16, 16)] + 1.0
        slice_v[pl.ds(j * 16, 16)] = _rsqrt16(deg)
    pltpu.sync_copy(slice_v, hist_sh.at[pl.ds(row0, RPT)])

    @pl.when(cid == 0)
    def _dis_out():
        pltpu.sync_copy(slice_v, dis_hbm.at[pl.ds(row0, RPT)])

    plsc.subcore_barrier()
    pltpu.sync_copy(hist_sh, dis_v)

    wid = cid * NS + sid
    mbase = wid * E_MSG

    def stage_idx(ci, bn):
        pltpu.async_copy(src_hbm.at[pl.ds(mbase + ci * C, C)],
                         src_bufs[bn], sem_idx[bn])
        pltpu.async_copy(dst_hbm.at[pl.ds(mbase + ci * C, C)],
                         dst_bufs[bn], sem_idx[bn])

    def chunk_body(ci, b3, b2, wait_scat, do_stage):
        bn = (b3 + 1) % 3
        if wait_scat:
            pltpu.make_async_copy(xw_hbm.at[pl.ds(0, C)], rows_bufs[b2],
                                  sem_scat[b2]).wait()
        if do_stage is True:
            stage_idx(ci + 1, bn)
        elif do_stage is not False:
            @pl.when(do_stage)
            def _():
                stage_idx(ci + 1, bn)
        pltpu.make_async_copy(src_hbm.at[pl.ds(0, C)], src_bufs[b3],
                              sem_idx[b3]).wait()
        pltpu.make_async_copy(dst_hbm.at[pl.ds(0, C)], dst_bufs[b3],
                              sem_idx[b3]).wait()
        gat = pltpu.async_copy(xw_hbm.at[src_bufs[b3]], rows_bufs[b2],
                               sem_rows[b2])
        for g in range(C // 16):
            sv = src_bufs[b3][pl.ds(g * 16, 16)]
            dv = dst_bufs[b3][pl.ds(g * 16, 16)]
            ns = plsc.load_gather(dis_v, [sv])
            nd = plsc.load_gather(dis_v, [dv])
            norm_bufs[b3][pl.ds(g * 16, 16)] = ns * nd
        gat.wait()

        rows = rows_bufs[b2]
        norm = norm_bufs[b3]

        @pl.loop(0, C, unroll=2)
        def _scale(r):
            idx = jnp.broadcast_to(r, (16,)).astype(jnp.int32)
            spl = plsc.load_gather(norm, [idx])
            for j in range(DIM // 16):
                rows[r, pl.ds(j * 16, 16)] = rows[r, pl.ds(j * 16, 16)] * spl

        pltpu.async_copy(rows_bufs[b2], acc_sh.at[dst_bufs[b3]],
                         sem_scat[b2], add=True)

    stage_idx(0, 0)
    for ci in range(6):
        chunk_body(ci, ci % 3, ci % 2, wait_scat=ci >= 2, do_stage=True)

    @pl.loop(1, MSG_CHUNKS // 6)
    def _msg(si):
        ci0 = si * 6
        for k in range(6):
            ci = ci0 + k
            chunk_body(ci, k % 3, k % 2, wait_scat=True,
                       do_stage=(ci + 1 < MSG_CHUNKS))

    pltpu.make_async_copy(xw_hbm.at[pl.ds(0, C)], rows_b0, sem_s0).wait()
    pltpu.make_async_copy(xw_hbm.at[pl.ds(0, C)], rows_b1, sem_s1).wait()

    tb = mbase + MSG_CHUNKS * C
    rowst = rows_b0.at[pl.ds(0, MSG_TAIL)]
    pltpu.sync_copy(src_hbm.at[pl.ds(tb, MSG_TAIL)], srct_v)
    pltpu.sync_copy(dst_hbm.at[pl.ds(tb, MSG_TAIL)], dstt_v)
    normt_v[...] = (plsc.load_gather(dis_v, [srct_v[...]]) *
                    plsc.load_gather(dis_v, [dstt_v[...]]))
    pltpu.async_copy(xw_hbm.at[srct_v], rowst, sem_x).wait()

    @pl.loop(0, MSG_TAIL)
    def _scale_tail(r):
        idx = jnp.broadcast_to(r, (16,)).astype(jnp.int32)
        spl = plsc.load_gather(normt_v, [idx])
        for j in range(DIM // 16):
            rows_b0[r, pl.ds(j * 16, 16)] = (
                rows_b0[r, pl.ds(j * 16, 16)] * spl)

    pltpu.sync_copy(rowst, acc_sh.at[dstt_v], add=True)
    plsc.subcore_barrier()

    pltpu.sync_copy(acc_sh.at[pl.ds(row0, RPT)],
                    part_hbm.at[cid, pl.ds(row0, RPT)])


_sc_kernel = functools.partial(
    pl.kernel,
    out_type=(
        jax.ShapeDtypeStruct((NC, NPAD, DIM), jnp.float32),
        jax.ShapeDtypeStruct((NPAD,), jnp.float32),
    ),
    mesh=plsc.VectorSubcoreMesh(core_axis_name="c", subcore_axis_name="s",
                                num_cores=NC, num_subcores=NS),
    compiler_params=pltpu.CompilerParams(needs_layout_passes=False),
    scratch_types=[
        pltpu.VMEM_SHARED((NPAD,), jnp.float32),
        pltpu.VMEM_SHARED((NPAD, DIM), jnp.float32),
        pltpu.VMEM((NPAD,), jnp.float32),
        pltpu.VMEM((C,), jnp.int32),
        pltpu.VMEM((C,), jnp.int32),
        pltpu.VMEM((C,), jnp.int32),
        pltpu.VMEM((C,), jnp.int32),
        pltpu.VMEM((C,), jnp.int32),
        pltpu.VMEM((C,), jnp.int32),
        pltpu.VMEM((C,), jnp.float32),
        pltpu.VMEM((C,), jnp.float32),
        pltpu.VMEM((C,), jnp.float32),
        pltpu.VMEM((C, DIM), jnp.float32),
        pltpu.VMEM((C, DIM), jnp.float32),
        pltpu.VMEM((C,), jnp.float32),
        pltpu.VMEM((DEG_TAIL,), jnp.float32),
        pltpu.VMEM((RPT,), jnp.float32),
        pltpu.VMEM((MSG_TAIL,), jnp.int32),
        pltpu.VMEM((MSG_TAIL,), jnp.int32),
        pltpu.VMEM((DEG_TAIL,), jnp.int32),
        pltpu.VMEM((MSG_TAIL,), jnp.float32),
        pltpu.SemaphoreType.DMA,
        pltpu.SemaphoreType.DMA,
        pltpu.SemaphoreType.DMA,
        pltpu.SemaphoreType.DMA,
        pltpu.SemaphoreType.DMA,
        pltpu.SemaphoreType.DMA,
        pltpu.SemaphoreType.DMA,
        pltpu.SemaphoreType.DMA,
        pltpu.SemaphoreType.DMA,
    ],
)(_sc_body)


def kernel(x, edge_index, W, b):
    ei = edge_index.astype(jnp.int32)
    src = ei[0]
    dst = ei[1]

    xw = pl.pallas_call(
        _mm_body,
        grid=(N_NODES // _MM_BLOCK,),
        in_specs=[
            pl.BlockSpec((_MM_BLOCK, DIM), lambda i: (i, 0)),
            pl.BlockSpec((DIM, DIM), lambda i: (0, 0)),
        ],
        out_specs=pl.BlockSpec((_MM_BLOCK, DIM), lambda i: (i, 0)),
        out_shape=jax.ShapeDtypeStruct((N_NODES, DIM), jnp.float32),
    )(x, W)

    part, dis = _sc_kernel(src, dst, xw)

    dis2d = dis[:N_NODES].reshape(N_NODES, 1)
    b2d = b.reshape(1, DIM)
    out = pl.pallas_call(
        _final_body,
        grid=(N_NODES // _MM_BLOCK,),
        in_specs=[
            pl.BlockSpec((1, _MM_BLOCK, DIM), lambda i: (0, i, 0)),
            pl.BlockSpec((1, _MM_BLOCK, DIM), lambda i: (1, i, 0)),
            pl.BlockSpec((_MM_BLOCK, DIM), lambda i: (i, 0)),
            pl.BlockSpec((_MM_BLOCK, 1), lambda i: (i, 0)),
            pl.BlockSpec((1, DIM), lambda i: (0, 0)),
        ],
        out_specs=pl.BlockSpec((_MM_BLOCK, DIM), lambda i: (i, 0)),
        out_shape=jax.ShapeDtypeStruct((N_NODES, DIM), jnp.float32),
    )(part, part, xw, dis2d, b2d)
    return out


# speedup vs baseline: 34.8504x; 1.4702x over previous
import functools

import jax
import jax.numpy as jnp
from jax import lax
from jax.experimental import pallas as pl
from jax.experimental.pallas import tpu as pltpu
from jax.experimental.pallas import tpu_sc as plsc

N_NODES = 10000
N_EDGES = 320000
DIM = 128

NC = 2
NS = 16
NW = NC * NS
NPAD = 10240
RPT = NPAD // NS
C = 128
E_TILE = N_EDGES // NW
CHUNKS = E_TILE // C
TAIL = E_TILE - CHUNKS * C

_MM_BLOCK = 400


def _xws_body(x_ref, w_ref, p0_ref, p1_ref, xws_ref, dis_ref):
    deg = p0_ref[0] + p1_ref[0] + 1.0
    dis = lax.rsqrt(deg)
    xw = jnp.dot(x_ref[...], w_ref[...], preferred_element_type=jnp.float32)
    dis_ref[...] = dis
    xws_ref[...] = dis * xw


def _final_body(p0_ref, p1_ref, xws_ref, dis_ref, b_ref, o_ref):
    acc = dis_ref[...] * (p0_ref[0] + p1_ref[0] + xws_ref[...]) + b_ref[...]
    o_ref[...] = jnp.maximum(acc, 0.0)


def _deg_body(dst_hbm, degp_hbm, hist_sh,
              dst_b0, dst_b1, dst_b2, ones_v, ones16_v, slice_v, dstt_v,
              sem_i0, sem_i1, sem_i2, sem_s0, sem_s1, sem_s2):
    cid = lax.axis_index("c")
    sid = lax.axis_index("s")
    z16 = jnp.zeros((16,), jnp.float32)
    o16 = jnp.ones((16,), jnp.float32)
    row0 = sid * RPT
    dst_bufs = [dst_b0, dst_b1, dst_b2]
    sem_idx = [sem_i0, sem_i1, sem_i2]
    sem_scat = [sem_s0, sem_s1, sem_s2]

    for j in range(C // 16):
        ones_v[pl.ds(j * 16, 16)] = o16
    ones16_v[...] = o16
    for j in range(RPT // 16):
        slice_v[pl.ds(j * 16, 16)] = z16
    pltpu.sync_copy(slice_v, hist_sh.at[pl.ds(row0, RPT)])
    plsc.subcore_barrier()

    base = (cid * NS + sid) * E_TILE

    def stage(ci, bn):
        pltpu.async_copy(dst_hbm.at[pl.ds(base + ci * C, C)],
                         dst_bufs[bn], sem_idx[bn])

    def body(ci, b, bn, wait_scat, do_stage):
        if wait_scat:
            pltpu.make_async_copy(dst_hbm.at[pl.ds(0, C)], dst_bufs[bn],
                                  sem_scat[bn]).wait()
        if do_stage is True:
            stage(ci + 1, bn)
        elif do_stage is not False:
            @pl.when(do_stage)
            def _():
                stage(ci + 1, bn)
        pltpu.make_async_copy(dst_hbm.at[pl.ds(0, C)], dst_bufs[b],
                              sem_idx[b]).wait()
        pltpu.async_copy(ones_v, hist_sh.at[dst_bufs[b]], sem_scat[b],
                         add=True)

    stage(0, 0)
    body(0, 0, 1, wait_scat=False, do_stage=True)
    body(1, 1, 2, wait_scat=False, do_stage=True)
    body(2, 2, 0, wait_scat=True, do_stage=True)

    @pl.loop(1, CHUNKS // 3)
    def _deg(si):
        ci0 = si * 3
        for k in range(3):
            ci = ci0 + k
            body(ci, k, (k + 1) % 3, wait_scat=True,
                 do_stage=ci + 1 < CHUNKS)

    for b in (1, 2):
        pltpu.make_async_copy(dst_hbm.at[pl.ds(0, C)], dst_bufs[b],
                              sem_scat[b]).wait()
    pltpu.sync_copy(dst_hbm.at[pl.ds(base + CHUNKS * C, TAIL)], dstt_v)
    pltpu.sync_copy(ones16_v, hist_sh.at[dstt_v], add=True)
    plsc.subcore_barrier()
    pltpu.sync_copy(hist_sh.at[pl.ds(row0, RPT)],
                    degp_hbm.at[cid, pl.ds(row0, RPT)])


_deg_kernel = functools.partial(
    pl.kernel,
    out_type=jax.ShapeDtypeStruct((NC, NPAD), jnp.float32),
    mesh=plsc.VectorSubcoreMesh(core_axis_name="c", subcore_axis_name="s",
                                num_cores=NC, num_subcores=NS),
    compiler_params=pltpu.CompilerParams(needs_layout_passes=False),
    scratch_types=[
        pltpu.VMEM_SHARED((NPAD,), jnp.float32),
        pltpu.VMEM((C,), jnp.int32),
        pltpu.VMEM((C,), jnp.int32),
        pltpu.VMEM((C,), jnp.int32),
        pltpu.VMEM((C,), jnp.float32),
        pltpu.VMEM((TAIL,), jnp.float32),
        pltpu.VMEM((RPT,), jnp.float32),
        pltpu.VMEM((TAIL,), jnp.int32),
        pltpu.SemaphoreType.DMA,
        pltpu.SemaphoreType.DMA,
        pltpu.SemaphoreType.DMA,
        pltpu.SemaphoreType.DMA,
        pltpu.SemaphoreType.DMA,
        pltpu.SemaphoreType.DMA,
    ],
)(_deg_body)


def _msg_body(src_hbm, dst_hbm, xws_hbm, part_hbm, acc_sh,
              src_b0, src_b1, src_b2, dst_b0, dst_b1, dst_b2,
              rows_b0, rows_b1, srct_v, dstt_v,
              sem_i0, sem_i1, sem_i2, sem_r0, sem_r1,
              sem_s0, sem_s1, sem_x):
    cid = lax.axis_index("c")
    sid = lax.axis_index("s")
    z16 = jnp.zeros((16,), jnp.float32)
    row0 = sid * RPT
    src_bufs = [src_b0, src_b1, src_b2]
    dst_bufs = [dst_b0, dst_b1, dst_b2]
    rows_bufs = [rows_b0, rows_b1]
    sem_idx = [sem_i0, sem_i1, sem_i2]
    sem_rows = [sem_r0, sem_r1]
    sem_scat = [sem_s0, sem_s1]

    @pl.loop(0, C)
    def _zero_rows(i):
        for j in range(DIM // 16):
            rows_b0[i, pl.ds(j * 16, 16)] = z16

    for k in range(RPT // C):
        pltpu.sync_copy(rows_b0, acc_sh.at[pl.ds(row0 + k * C, C)])
    plsc.subcore_barrier()

    base = (cid * NS + sid) * E_TILE

    def stage(ci, bn):
        pltpu.async_copy(src_hbm.at[pl.ds(base + ci * C, C)],
                         src_bufs[bn], sem_idx[bn])
        pltpu.async_copy(dst_hbm.at[pl.ds(base + ci * C, C)],
                         dst_bufs[bn], sem_idx[bn])

    def body(ci, b3, b2, wait_scat, do_stage):
        bn = (b3 + 1) % 3
        if wait_scat:
            pltpu.make_async_copy(xws_hbm.at[pl.ds(0, C)], rows_bufs[b2],
                                  sem_scat[b2]).wait()
        if do_stage is True:
            stage(ci + 1, bn)
        elif do_stage is not False:
            @pl.when(do_stage)
            def _():
                stage(ci + 1, bn)
        pltpu.make_async_copy(src_hbm.at[pl.ds(0, C)], src_bufs[b3],
                              sem_idx[b3]).wait()
        pltpu.make_async_copy(dst_hbm.at[pl.ds(0, C)], dst_bufs[b3],
                              sem_idx[b3]).wait()
        pltpu.async_copy(xws_hbm.at[src_bufs[b3]], rows_bufs[b2],
                         sem_rows[b2]).wait()
        pltpu.async_copy(rows_bufs[b2], acc_sh.at[dst_bufs[b3]],
                         sem_scat[b2], add=True)

    stage(0, 0)
    for ci in range(6):
        body(ci, ci % 3, ci % 2, wait_scat=ci >= 2, do_stage=True)

    @pl.loop(1, CHUNKS // 6)
    def _msg(si):
        ci0 = si * 6
        for k in range(6):
            ci = ci0 + k
            body(ci, k % 3, k % 2, wait_scat=True,
                 do_stage=ci + 1 < CHUNKS)

    pltpu.make_async_copy(xws_hbm.at[pl.ds(0, C)], rows_b0, sem_s0).wait()
    pltpu.make_async_copy(xws_hbm.at[pl.ds(0, C)], rows_b1, sem_s1).wait()

    tb = base + CHUNKS * C
    rowst = rows_b0.at[pl.ds(0, TAIL)]
    pltpu.sync_copy(src_hbm.at[pl.ds(tb, TAIL)], srct_v)
    pltpu.sync_copy(dst_hbm.at[pl.ds(tb, TAIL)], dstt_v)
    pltpu.async_copy(xws_hbm.at[srct_v], rowst, sem_x).wait()
    pltpu.sync_copy(rowst, acc_sh.at[dstt_v], add=True)
    plsc.subcore_barrier()

    pltpu.sync_copy(acc_sh.at[pl.ds(row0, RPT)],
                    part_hbm.at[cid, pl.ds(row0, RPT)])


_msg_kernel = functools.partial(
    pl.kernel,
    out_type=jax.ShapeDtypeStruct((NC, NPAD, DIM), jnp.float32),
    mesh=plsc.VectorSubcoreMesh(core_axis_name="c", subcore_axis_name="s",
                                num_cores=NC, num_subcores=NS),
    compiler_params=pltpu.CompilerParams(needs_layout_passes=False),
    scratch_types=[
        pltpu.VMEM_SHARED((NPAD, DIM), jnp.float32),
        pltpu.VMEM((C,), jnp.int32),
        pltpu.VMEM((C,), jnp.int32),
        pltpu.VMEM((C,), jnp.int32),
        pltpu.VMEM((C,), jnp.int32),
        pltpu.VMEM((C,), jnp.int32),
        pltpu.VMEM((C,), jnp.int32),
        pltpu.VMEM((C, DIM), jnp.float32),
        pltpu.VMEM((C, DIM), jnp.float32),
        pltpu.VMEM((TAIL,), jnp.int32),
        pltpu.VMEM((TAIL,), jnp.int32),
        pltpu.SemaphoreType.DMA,
        pltpu.SemaphoreType.DMA,
        pltpu.SemaphoreType.DMA,
        pltpu.SemaphoreType.DMA,
        pltpu.SemaphoreType.DMA,
        pltpu.SemaphoreType.DMA,
        pltpu.SemaphoreType.DMA,
        pltpu.SemaphoreType.DMA,
    ],
)(_msg_body)


def kernel(x, edge_index, W, b):
    ei = edge_index.astype(jnp.int32)
    src = ei[0]
    dst = ei[1]

    degp = _deg_kernel(dst)
    degp3 = degp[:, :N_NODES].reshape(NC, N_NODES, 1)

    xws, dis2d = pl.pallas_call(
        _xws_body,
        grid=(N_NODES // _MM_BLOCK,),
        in_specs=[
            pl.BlockSpec((_MM_BLOCK, DIM), lambda i: (i, 0)),
            pl.BlockSpec((DIM, DIM), lambda i: (0, 0)),
            pl.BlockSpec((1, _MM_BLOCK, 1), lambda i: (0, i, 0)),
            pl.BlockSpec((1, _MM_BLOCK, 1), lambda i: (1, i, 0)),
        ],
        out_specs=[
            pl.BlockSpec((_MM_BLOCK, DIM), lambda i: (i, 0)),
            pl.BlockSpec((_MM_BLOCK, 1), lambda i: (i, 0)),
        ],
        out_shape=[
            jax.ShapeDtypeStruct((N_NODES, DIM), jnp.float32),
            jax.ShapeDtypeStruct((N_NODES, 1), jnp.float32),
        ],
    )(x, W, degp3, degp3)

    part = _msg_kernel(src, dst, xws)

    b2d = b.reshape(1, DIM)
    out = pl.pallas_call(
        _final_body,
        grid=(N_NODES // _MM_BLOCK,),
        in_specs=[
            pl.BlockSpec((1, _MM_BLOCK, DIM), lambda i: (0, i, 0)),
            pl.BlockSpec((1, _MM_BLOCK, DIM), lambda i: (1, i, 0)),
            pl.BlockSpec((_MM_BLOCK, DIM), lambda i: (i, 0)),
            pl.BlockSpec((_MM_BLOCK, 1), lambda i: (i, 0)),
            pl.BlockSpec((1, DIM), lambda i: (0, 0)),
        ],
        out_specs=pl.BlockSpec((_MM_BLOCK, DIM), lambda i: (i, 0)),
        out_shape=jax.ShapeDtypeStruct((N_NODES, DIM), jnp.float32),
    )(part, part, xws, dis2d, b2d)
    return out
